# baseline (speedup 1.0000x reference)
import functools

import jax
import jax.numpy as jnp
from jax import lax
from jax.experimental import pallas as pl
from jax.experimental.pallas import tpu as pltpu
from jax.experimental.pallas import tpu_sc as plsc

E = 8
K = 2
D = 1024
F = 2048
M = 2048

T = 256
NP = 6144
NB = NP // T

NC = 2
NS = 16
NW = NC * NS


def _sc_gather(table, idx, n_chunks):
    R, Dd = table.shape
    B = idx.shape[0]
    b_per_w = B // NW
    ch = b_per_w // n_chunks
    mesh = plsc.VectorSubcoreMesh(
        core_axis_name="c", subcore_axis_name="s", num_cores=NC, num_subcores=NS
    )

    @functools.partial(
        pl.kernel,
        out_type=jax.ShapeDtypeStruct((B, Dd), table.dtype),
        mesh=mesh,
        scratch_types=[
            pltpu.VMEM((n_chunks, ch), jnp.int32),
            pltpu.VMEM((ch, Dd), table.dtype),
            pltpu.SemaphoreType.DMA,
        ],
    )
    def k(table_hbm, idx_hbm, out_hbm, idx_v, rows_v, sem):
        wid = lax.axis_index("s") * NC + lax.axis_index("c")
        base = wid * b_per_w
        for c in range(n_chunks):
            pltpu.sync_copy(idx_hbm.at[pl.ds(base + c * ch, ch)], idx_v.at[c])
            pltpu.async_copy(table_hbm.at[idx_v.at[c]], rows_v, sem).wait()
            pltpu.sync_copy(rows_v, out_hbm.at[pl.ds(base + c * ch, ch)])

    return k(table, idx)


def _tc_gmm(xs, w1, w2, w3, block_expert):
    def body(be_ref, x_ref, w1_ref, w3_ref, w2_ref, o_ref):
        x = x_ref[...].astype(jnp.bfloat16)
        h = jnp.dot(x, w1_ref[0].astype(jnp.bfloat16), preferred_element_type=jnp.float32)
        g = jnp.dot(x, w3_ref[0].astype(jnp.bfloat16), preferred_element_type=jnp.float32)
        a = (h * jax.nn.sigmoid(h) * g).astype(jnp.bfloat16)
        o_ref[...] = jnp.dot(a, w2_ref[0].astype(jnp.bfloat16), preferred_element_type=jnp.float32)

    grid_spec = pltpu.PrefetchScalarGridSpec(
        num_scalar_prefetch=1,
        grid=(NB,),
        in_specs=[
            pl.BlockSpec((T, D), lambda b, be: (b, 0)),
            pl.BlockSpec((1, D, F), lambda b, be: (be[b], 0, 0)),
            pl.BlockSpec((1, D, F), lambda b, be: (be[b], 0, 0)),
            pl.BlockSpec((1, F, D), lambda b, be: (be[b], 0, 0)),
        ],
        out_specs=pl.BlockSpec((T, D), lambda b, be: (b, 0)),
    )
    return pl.pallas_call(
        body,
        grid_spec=grid_spec,
        out_shape=jax.ShapeDtypeStruct((NP, D), jnp.float32),
    )(block_expert, xs, w1, w3, w2)


def _route(top_ks):
    top_flat = top_ks.reshape(-1).astype(jnp.int32)
    counts = jnp.zeros((E,), jnp.int32).at[top_flat].add(1)
    padded = ((counts + T - 1) // T) * T
    offs_p = jnp.concatenate(
        [jnp.zeros((1,), jnp.int32), jnp.cumsum(padded)[:-1]]
    )
    offs_u = jnp.concatenate(
        [jnp.zeros((1,), jnp.int32), jnp.cumsum(counts)[:-1]]
    )
    order = jnp.argsort(top_flat, stable=True).astype(jnp.int32)
    eid = top_flat[order]
    dest = offs_p[eid] + (jnp.arange(M * K, dtype=jnp.int32) - offs_u[eid])
    sidx = jnp.zeros((NP,), jnp.int32).at[dest].set(order // K)
    pos = jnp.zeros((M * K,), jnp.int32).at[order].set(dest)
    b_idx = jnp.arange(NB, dtype=jnp.int32)
    be = (
        jnp.sum((b_idx[None, :] >= (offs_p // T)[:, None]).astype(jnp.int32), axis=0)
        - 1
    )
    return sidx, pos, be.astype(jnp.int32)


def kernel(hidden_states, top_ks, w1, w2, w3):
    sidx, pos, be = _route(top_ks)
    xs = _sc_gather(hidden_states, sidx, n_chunks=4)
    ys = _tc_gmm(xs, w1, w2, w3, be)
    out = _sc_gather(ys, pos, n_chunks=2)
    return out.reshape(M, K, D)


# speedup vs baseline: 1.4000x; 1.4000x over previous
import functools

import jax
import jax.numpy as jnp
from jax import lax
from jax.experimental import pallas as pl
from jax.experimental.pallas import tpu as pltpu
from jax.experimental.pallas import tpu_sc as plsc

E = 8
K = 2
D = 1024
F = 2048
M = 2048

T = 256
NP = 6144
NB = NP // T

NC = 2
NS = 16
NW = NC * NS


def _sc_gather(table, idx, n_chunks):
    R, Dd = table.shape
    B = idx.shape[0]
    b_per_w = B // NW
    ch = b_per_w // n_chunks
    mesh = plsc.VectorSubcoreMesh(
        core_axis_name="c", subcore_axis_name="s", num_cores=NC, num_subcores=NS
    )

    @functools.partial(
        pl.kernel,
        out_type=jax.ShapeDtypeStruct((B, Dd), table.dtype),
        mesh=mesh,
        scratch_types=[
            pltpu.VMEM((n_chunks, ch), jnp.int32),
            pltpu.VMEM((ch, Dd), table.dtype),
            pltpu.SemaphoreType.DMA,
        ],
    )
    def k(table_hbm, idx_hbm, out_hbm, idx_v, rows_v, sem):
        wid = lax.axis_index("s") * NC + lax.axis_index("c")
        base = wid * b_per_w
        for c in range(n_chunks):
            pltpu.sync_copy(idx_hbm.at[pl.ds(base + c * ch, ch)], idx_v.at[c])
            pltpu.async_copy(table_hbm.at[idx_v.at[c]], rows_v, sem).wait()
            pltpu.sync_copy(rows_v, out_hbm.at[pl.ds(base + c * ch, ch)])

    return k(table, idx)


def _tc_gmm(xs, w1, w2, w3, block_expert):
    def body(be_ref, x_ref, w1_ref, w3_ref, w2_ref, o_ref):
        x = x_ref[...].astype(jnp.bfloat16)
        h = jnp.dot(x, w1_ref[0].astype(jnp.bfloat16), preferred_element_type=jnp.float32)
        g = jnp.dot(x, w3_ref[0].astype(jnp.bfloat16), preferred_element_type=jnp.float32)
        a = (h * jax.nn.sigmoid(h) * g).astype(jnp.bfloat16)
        o_ref[...] = jnp.dot(a, w2_ref[0].astype(jnp.bfloat16), preferred_element_type=jnp.float32)

    grid_spec = pltpu.PrefetchScalarGridSpec(
        num_scalar_prefetch=1,
        grid=(NB,),
        in_specs=[
            pl.BlockSpec((T, D), lambda b, be: (b, 0)),
            pl.BlockSpec((1, D, F), lambda b, be: (be[b], 0, 0)),
            pl.BlockSpec((1, D, F), lambda b, be: (be[b], 0, 0)),
            pl.BlockSpec((1, F, D), lambda b, be: (be[b], 0, 0)),
        ],
        out_specs=pl.BlockSpec((T, D), lambda b, be: (b, 0)),
    )
    return pl.pallas_call(
        body,
        grid_spec=grid_spec,
        out_shape=jax.ShapeDtypeStruct((NP, D), jnp.float32),
    )(block_expert, xs, w1, w3, w2)


def _route(top_ks):
    top_flat = top_ks.reshape(-1).astype(jnp.int32)
    counts = jnp.zeros((E,), jnp.int32).at[top_flat].add(1)
    padded = ((counts + T - 1) // T) * T
    offs_p = jnp.concatenate(
        [jnp.zeros((1,), jnp.int32), jnp.cumsum(padded)[:-1]]
    )
    offs_u = jnp.concatenate(
        [jnp.zeros((1,), jnp.int32), jnp.cumsum(counts)[:-1]]
    )
    order = jnp.argsort(top_flat, stable=True).astype(jnp.int32)
    eid = top_flat[order]
    dest = offs_p[eid] + (jnp.arange(M * K, dtype=jnp.int32) - offs_u[eid])
    sidx = (jnp.arange(NP, dtype=jnp.int32) % M).at[dest].set(order // K)
    pos = jnp.zeros((M * K,), jnp.int32).at[order].set(dest)
    b_idx = jnp.arange(NB, dtype=jnp.int32)
    be = (
        jnp.sum((b_idx[None, :] >= (offs_p // T)[:, None]).astype(jnp.int32), axis=0)
        - 1
    )
    return sidx, pos, be.astype(jnp.int32)


def kernel(hidden_states, top_ks, w1, w2, w3):
    sidx, pos, be = _route(top_ks)
    xs = _sc_gather(hidden_states, sidx, n_chunks=4)
    ys = _tc_gmm(xs, w1, w2, w3, be)
    out = _sc_gather(ys, pos, n_chunks=2)
    return out.reshape(M, K, D)


# speedup vs baseline: 1.6885x; 1.2061x over previous
import functools

import jax
import jax.numpy as jnp
from jax import lax
from jax.experimental import pallas as pl
from jax.experimental.pallas import tpu as pltpu
from jax.experimental.pallas import tpu_sc as plsc

E = 8
K = 2
D = 1024
F = 2048
M = 2048

T = 256
NP = 6144
NB = NP // T

NC = 2
NS = 16
NW = NC * NS


def _sc_gather(table, idx, n_chunks):
    R, Dd = table.shape
    B = idx.shape[0]
    b_per_w = B // NW
    ch = b_per_w // n_chunks
    mesh = plsc.VectorSubcoreMesh(
        core_axis_name="c", subcore_axis_name="s", num_cores=NC, num_subcores=NS
    )

    @functools.partial(
        pl.kernel,
        out_type=jax.ShapeDtypeStruct((B, Dd), table.dtype),
        mesh=mesh,
        scratch_types=[
            pltpu.VMEM((n_chunks, ch), jnp.int32),
            pltpu.VMEM((ch, Dd), table.dtype),
            pltpu.SemaphoreType.DMA,
        ],
    )
    def k(table_hbm, idx_hbm, out_hbm, idx_v, rows_v, sem):
        wid = lax.axis_index("s") * NC + lax.axis_index("c")
        base = wid * b_per_w
        for c in range(n_chunks):
            pltpu.sync_copy(idx_hbm.at[pl.ds(base + c * ch, ch)], idx_v.at[c])
            pltpu.async_copy(table_hbm.at[idx_v.at[c]], rows_v, sem).wait()
            pltpu.sync_copy(rows_v, out_hbm.at[pl.ds(base + c * ch, ch)])

    return k(table, idx)


def _tc_gmm(xs, w1, w2, w3, block_expert):
    def body(be_ref, x_ref, w1_ref, w3_ref, w2_ref, o_ref):
        x = x_ref[...].astype(jnp.bfloat16)
        h = jnp.dot(x, w1_ref[0].astype(jnp.bfloat16), preferred_element_type=jnp.float32)
        g = jnp.dot(x, w3_ref[0].astype(jnp.bfloat16), preferred_element_type=jnp.float32)
        a = (h * jax.nn.sigmoid(h) * g).astype(jnp.bfloat16)
        o_ref[...] = jnp.dot(a, w2_ref[0].astype(jnp.bfloat16), preferred_element_type=jnp.float32)

    grid_spec = pltpu.PrefetchScalarGridSpec(
        num_scalar_prefetch=1,
        grid=(NB,),
        in_specs=[
            pl.BlockSpec((T, D), lambda b, be: (b, 0)),
            pl.BlockSpec((1, D, F), lambda b, be: (be[b], 0, 0)),
            pl.BlockSpec((1, D, F), lambda b, be: (be[b], 0, 0)),
            pl.BlockSpec((1, F, D), lambda b, be: (be[b], 0, 0)),
        ],
        out_specs=pl.BlockSpec((T, D), lambda b, be: (b, 0)),
    )
    return pl.pallas_call(
        body,
        grid_spec=grid_spec,
        out_shape=jax.ShapeDtypeStruct((NP, D), jnp.float32),
    )(block_expert, xs, w1, w3, w2)


def _sc_route_dispatch(hidden_states, top_flat):
    MK = M * K
    jpw = MK // NW
    half = jpw // 2
    nvr = MK // 16
    vpw = jpw // 16
    mesh = plsc.VectorSubcoreMesh(
        core_axis_name="c", subcore_axis_name="s", num_cores=NC, num_subcores=NS
    )

    @functools.partial(
        pl.kernel,
        out_type=(
            jax.ShapeDtypeStruct((NP, D), jnp.float32),
            jax.ShapeDtypeStruct((MK,), jnp.int32),
            jax.ShapeDtypeStruct((32,), jnp.int32),
        ),
        mesh=mesh,
        scratch_types=[
            pltpu.VMEM((MK,), jnp.int32),
            pltpu.VMEM((jpw,), jnp.int32),
            pltpu.VMEM((2, half), jnp.int32),
            pltpu.VMEM((2, half), jnp.int32),
            pltpu.VMEM((half, D), jnp.float32),
            pltpu.SemaphoreType.DMA,
            pltpu.SemaphoreType.DMA,
        ],
    )
    def k(hs, tf, disp, pos, be, ids, stage, idxg, idxs, rows, sem1, sem2):
        wid = lax.axis_index("s") * NC + lax.axis_index("c")
        pltpu.sync_copy(tf, ids)
        iota = lax.iota(jnp.int32, 16)
        lane15 = jnp.full((16,), 15, jnp.int32)
        zero16 = jnp.zeros((16,), jnp.int32)

        gdn = lax.GatherDimensionNumbers(
            offset_dims=(), collapsed_slice_dims=(0,), start_index_map=(0,)
        )

        def splat(vec, idxv):
            return lax.gather(
                vec,
                idxv.reshape(16, 1),
                gdn,
                (1,),
                mode=lax.GatherScatterMode.PROMISE_IN_BOUNDS,
            )

        one16 = jnp.full((16,), 1, jnp.int32)

        def butterfly_sum(x):
            for s in (1, 2, 4, 8):
                x = x + splat(x, jnp.bitwise_xor(iota, s))
            return x

        def vcumsum(x):
            for s in (1, 2, 4, 8):
                shifted = splat(x, jnp.maximum(iota - s, 0))
                x = x + jnp.where(iota >= s, shifted, zero16)
            return x

        def chunk_hist(w0, acc):
            def pb(j, accs):
                a1, a2 = accs
                v = ids[pl.ds((w0 * vpw + j) * 16, 16)]
                sh = jnp.left_shift(one16, (v & 3) * 8)
                a1 = a1 + jnp.where(v < 4, sh, zero16)
                a2 = a2 + jnp.where(v >= 4, sh, zero16)
                return a1, a2

            a1, a2 = lax.fori_loop(0, vpw, pb, (zero16, zero16))
            t1 = butterfly_sum(a1)
            t2 = butterfly_sum(a2)
            for e in range(4):
                c1 = jnp.bitwise_and(jnp.right_shift(t1, e * 8), 255)
                c2 = jnp.bitwise_and(jnp.right_shift(t2, e * 8), 255)
                acc = (
                    acc
                    + jnp.where(iota == e, c1, zero16)
                    + jnp.where(iota == e + 4, c2, zero16)
                )
            return acc

        tot = lax.fori_loop(0, NW, chunk_hist, zero16)
        prior = lax.fori_loop(0, wid, chunk_hist, zero16)

        padded = jnp.bitwise_and(tot + (T - 1), -T)
        offs_p = vcumsum(padded) - padded
        base_vec = offs_p + prior

        @pl.when(wid == 0)
        def _():
            for r in range(2):
                bstart = (iota + 16 * r) * T
                accb = jnp.full((16,), -1, jnp.int32)
                for e in range(E):
                    offe = splat(offs_p, jnp.full((16,), e, jnp.int32))
                    accb = accb + jnp.where(bstart >= offe, 1, 0)
                stage[pl.ds(16 * r, 16)] = accb
            pltpu.sync_copy(stage.at[pl.ds(0, 32)], be)

        for i in range(vpw):
            v = ids[pl.ds(wid * jpw + 16 * i, 16)]
            dest = jnp.zeros((16,), jnp.int32)
            for e in range(E):
                m = v == e
                cs = vcumsum(jnp.where(m, 1, 0))
                bse = splat(base_vec, jnp.full((16,), e, jnp.int32))
                dest = jnp.where(m, bse + cs - 1, dest)
                tot_splat = splat(cs, lane15)
                base_vec = base_vec + jnp.where(iota == e, tot_splat, zero16)
            stage[pl.ds(16 * i, 16)] = dest
            idxs[i // (vpw // 2), pl.ds((i % (vpw // 2)) * 16, 16)] = dest
            idxg[i // (vpw // 2), pl.ds((i % (vpw // 2)) * 16, 16)] = jnp.right_shift(
                wid * jpw + 16 * i + iota, 1
            )
        pltpu.sync_copy(stage, pos.at[pl.ds(wid * jpw, jpw)])

        for c2 in range(2):
            pltpu.async_copy(hs.at[idxg.at[c2]], rows, sem1).wait()
            pltpu.async_copy(rows, disp.at[idxs.at[c2]], sem2).wait()

    return k(hidden_states, top_flat)


def kernel(hidden_states, top_ks, w1, w2, w3):
    top_flat = top_ks.reshape(-1).astype(jnp.int32)
    disp, pos, be = _sc_route_dispatch(hidden_states, top_flat)
    ys = _tc_gmm(disp, w1, w2, w3, be)
    out = _sc_gather(ys, pos, n_chunks=2)
    return out.reshape(M, K, D)


# speedup vs baseline: 1.8424x; 1.0912x over previous
import functools

import jax
import jax.numpy as jnp
from jax import lax
from jax.experimental import pallas as pl
from jax.experimental.pallas import tpu as pltpu
from jax.experimental.pallas import tpu_sc as plsc

E = 8
K = 2
D = 1024
F = 2048
M = 2048

T = 256
NP = 6144
NB = NP // T

NC = 2
NS = 16
NW = NC * NS


def _sc_gather(table, idx, n_chunks):
    R, Dd = table.shape
    B = idx.shape[0]
    b_per_w = B // NW
    ch = b_per_w // n_chunks
    mesh = plsc.VectorSubcoreMesh(
        core_axis_name="c", subcore_axis_name="s", num_cores=NC, num_subcores=NS
    )

    @functools.partial(
        pl.kernel,
        out_type=jax.ShapeDtypeStruct((B, Dd), table.dtype),
        mesh=mesh,
        scratch_types=[
            pltpu.VMEM((n_chunks, ch), jnp.int32),
            pltpu.VMEM((ch, Dd), table.dtype),
            pltpu.SemaphoreType.DMA,
        ],
    )
    def k(table_hbm, idx_hbm, out_hbm, idx_v, rows_v, sem):
        wid = lax.axis_index("s") * NC + lax.axis_index("c")
        base = wid * b_per_w
        for c in range(n_chunks):
            pltpu.sync_copy(idx_hbm.at[pl.ds(base + c * ch, ch)], idx_v.at[c])
            pltpu.async_copy(table_hbm.at[idx_v.at[c]], rows_v, sem).wait()
            pltpu.sync_copy(rows_v, out_hbm.at[pl.ds(base + c * ch, ch)])

    return k(table, idx)


def _tc_gmm(xs, w1, w2, w3, be32):
    beN = be32[:NB]
    change = jnp.concatenate(
        [jnp.ones((1,), jnp.int32), (beN[1:] != beN[:-1]).astype(jnp.int32)]
    )
    runid = jnp.cumsum(change) - 1
    slot = (runid % 2).astype(jnp.int32)
    big = jnp.int32(2 * NB)
    arrb = jnp.where(change == 1, jnp.arange(NB, dtype=jnp.int32), big)
    rcmin = jnp.flip(lax.cummin(jnp.flip(arrb)))
    ncp = jnp.concatenate([rcmin[1:], jnp.full((1,), big, jnp.int32)])
    has_next = (ncp < NB).astype(jnp.int32)
    nxt = beN[jnp.clip(ncp, 0, NB - 1)]

    def body(
        be_r, slot_r, nxt_r, hn_r, chg_r,
        x_ref, w1_hbm, w3_hbm, w2_hbm, o_ref,
        w1b, w3b, w2b, sems,
    ):
        b = pl.program_id(0)
        s = slot_r[b]

        def issue(e, sl):
            pltpu.make_async_copy(w1_hbm.at[e], w1b.at[sl], sems.at[sl]).start()
            pltpu.make_async_copy(w3_hbm.at[e], w3b.at[sl], sems.at[sl]).start()
            pltpu.make_async_copy(w2_hbm.at[e], w2b.at[sl], sems.at[sl]).start()

        def wait(sl):
            pltpu.make_async_copy(w1_hbm.at[0], w1b.at[sl], sems.at[sl]).wait()
            pltpu.make_async_copy(w3_hbm.at[0], w3b.at[sl], sems.at[sl]).wait()
            pltpu.make_async_copy(w2_hbm.at[0], w2b.at[sl], sems.at[sl]).wait()

        @pl.when(b == 0)
        def _():
            issue(be_r[0], s)

        @pl.when(chg_r[b] == 1)
        def _():
            wait(s)

            @pl.when(hn_r[b] == 1)
            def _():
                issue(nxt_r[b], 1 - s)

        x = x_ref[...].astype(jnp.bfloat16)
        h = jnp.dot(x, w1b[s].astype(jnp.bfloat16), preferred_element_type=jnp.float32)
        g = jnp.dot(x, w3b[s].astype(jnp.bfloat16), preferred_element_type=jnp.float32)
        a = (h * jax.nn.sigmoid(h) * g).astype(jnp.bfloat16)
        o_ref[...] = jnp.dot(a, w2b[s].astype(jnp.bfloat16), preferred_element_type=jnp.float32)

    grid_spec = pltpu.PrefetchScalarGridSpec(
        num_scalar_prefetch=5,
        grid=(NB,),
        in_specs=[
            pl.BlockSpec((T, D), lambda b, *_: (b, 0)),
            pl.BlockSpec(memory_space=pl.ANY),
            pl.BlockSpec(memory_space=pl.ANY),
            pl.BlockSpec(memory_space=pl.ANY),
        ],
        out_specs=pl.BlockSpec((T, D), lambda b, *_: (b, 0)),
        scratch_shapes=[
            pltpu.VMEM((2, D, F), jnp.float32),
            pltpu.VMEM((2, D, F), jnp.float32),
            pltpu.VMEM((2, F, D), jnp.float32),
            pltpu.SemaphoreType.DMA((2,)),
        ],
    )
    return pl.pallas_call(
        body,
        grid_spec=grid_spec,
        out_shape=jax.ShapeDtypeStruct((NP, D), jnp.float32),
    )(beN, slot, nxt, has_next, change, xs, w1, w3, w2)


def _sc_route_dispatch(hidden_states, top_flat):
    MK = M * K
    jpw = MK // NW
    half = jpw // 2
    nvr = MK // 16
    vpw = jpw // 16
    mesh = plsc.VectorSubcoreMesh(
        core_axis_name="c", subcore_axis_name="s", num_cores=NC, num_subcores=NS
    )

    @functools.partial(
        pl.kernel,
        out_type=(
            jax.ShapeDtypeStruct((NP, D), jnp.float32),
            jax.ShapeDtypeStruct((MK,), jnp.int32),
            jax.ShapeDtypeStruct((32,), jnp.int32),
        ),
        mesh=mesh,
        scratch_types=[
            pltpu.VMEM((MK,), jnp.int32),
            pltpu.VMEM((jpw,), jnp.int32),
            pltpu.VMEM((2, half), jnp.int32),
            pltpu.VMEM((2, half), jnp.int32),
            pltpu.VMEM((half, D), jnp.float32),
            pltpu.SemaphoreType.DMA,
            pltpu.SemaphoreType.DMA,
        ],
    )
    def k(hs, tf, disp, pos, be, ids, stage, idxg, idxs, rows, sem1, sem2):
        wid = lax.axis_index("s") * NC + lax.axis_index("c")
        pltpu.sync_copy(tf, ids)
        iota = lax.iota(jnp.int32, 16)
        lane15 = jnp.full((16,), 15, jnp.int32)
        zero16 = jnp.zeros((16,), jnp.int32)

        gdn = lax.GatherDimensionNumbers(
            offset_dims=(), collapsed_slice_dims=(0,), start_index_map=(0,)
        )

        def splat(vec, idxv):
            return lax.gather(
                vec,
                idxv.reshape(16, 1),
                gdn,
                (1,),
                mode=lax.GatherScatterMode.PROMISE_IN_BOUNDS,
            )

        one16 = jnp.full((16,), 1, jnp.int32)

        def butterfly_sum(x):
            for s in (1, 2, 4, 8):
                x = x + splat(x, jnp.bitwise_xor(iota, s))
            return x

        def vcumsum(x):
            for s in (1, 2, 4, 8):
                shifted = splat(x, jnp.maximum(iota - s, 0))
                x = x + jnp.where(iota >= s, shifted, zero16)
            return x

        def chunk_hist(w0, acc):
            def pb(j, accs):
                a1, a2 = accs
                v = ids[pl.ds((w0 * vpw + j) * 16, 16)]
                sh = jnp.left_shift(one16, (v & 3) * 8)
                a1 = a1 + jnp.where(v < 4, sh, zero16)
                a2 = a2 + jnp.where(v >= 4, sh, zero16)
                return a1, a2

            a1, a2 = lax.fori_loop(0, vpw, pb, (zero16, zero16))
            t1 = butterfly_sum(a1)
            t2 = butterfly_sum(a2)
            for e in range(4):
                c1 = jnp.bitwise_and(jnp.right_shift(t1, e * 8), 255)
                c2 = jnp.bitwise_and(jnp.right_shift(t2, e * 8), 255)
                acc = (
                    acc
                    + jnp.where(iota == e, c1, zero16)
                    + jnp.where(iota == e + 4, c2, zero16)
                )
            return acc

        tot = lax.fori_loop(0, NW, chunk_hist, zero16)
        prior = lax.fori_loop(0, wid, chunk_hist, zero16)

        padded = jnp.bitwise_and(tot + (T - 1), -T)
        offs_p = vcumsum(padded) - padded
        base_vec = offs_p + prior

        @pl.when(wid == 0)
        def _():
            for r in range(2):
                bstart = (iota + 16 * r) * T
                accb = jnp.full((16,), -1, jnp.int32)
                for e in range(E):
                    offe = splat(offs_p, jnp.full((16,), e, jnp.int32))
                    accb = accb + jnp.where(bstart >= offe, 1, 0)
                stage[pl.ds(16 * r, 16)] = accb
            pltpu.sync_copy(stage.at[pl.ds(0, 32)], be)

        for i in range(vpw):
            v = ids[pl.ds(wid * jpw + 16 * i, 16)]
            dest = jnp.zeros((16,), jnp.int32)
            for e in range(E):
                m = v == e
                cs = vcumsum(jnp.where(m, 1, 0))
                bse = splat(base_vec, jnp.full((16,), e, jnp.int32))
                dest = jnp.where(m, bse + cs - 1, dest)
                tot_splat = splat(cs, lane15)
                base_vec = base_vec + jnp.where(iota == e, tot_splat, zero16)
            stage[pl.ds(16 * i, 16)] = dest
            idxs[i // (vpw // 2), pl.ds((i % (vpw // 2)) * 16, 16)] = dest
            idxg[i // (vpw // 2), pl.ds((i % (vpw // 2)) * 16, 16)] = jnp.right_shift(
                wid * jpw + 16 * i + iota, 1
            )
        pltpu.sync_copy(stage, pos.at[pl.ds(wid * jpw, jpw)])

        for c2 in range(2):
            pltpu.async_copy(hs.at[idxg.at[c2]], rows, sem1).wait()
            pltpu.async_copy(rows, disp.at[idxs.at[c2]], sem2).wait()

    return k(hidden_states, top_flat)


def kernel(hidden_states, top_ks, w1, w2, w3):
    top_flat = top_ks.reshape(-1).astype(jnp.int32)
    disp, pos, be = _sc_route_dispatch(hidden_states, top_flat)
    ys = _tc_gmm(disp, w1, w2, w3, be)
    out = _sc_gather(ys, pos, n_chunks=2)
    return out.reshape(M, K, D)


# speedup vs baseline: 1.8503x; 1.0042x over previous
import functools

import jax
import jax.numpy as jnp
from jax import lax
from jax.experimental import pallas as pl
from jax.experimental.pallas import tpu as pltpu
from jax.experimental.pallas import tpu_sc as plsc

E = 8
K = 2
D = 1024
F = 2048
M = 2048

T = 256
NP = 6144
NB = NP // T

NC = 2
NS = 16
NW = NC * NS


def _sc_gather(table, idx, n_chunks):
    R, Dd = table.shape
    B = idx.shape[0]
    b_per_w = B // NW
    ch = b_per_w // n_chunks
    mesh = plsc.VectorSubcoreMesh(
        core_axis_name="c", subcore_axis_name="s", num_cores=NC, num_subcores=NS
    )

    @functools.partial(
        pl.kernel,
        out_type=jax.ShapeDtypeStruct((B, Dd), table.dtype),
        mesh=mesh,
        scratch_types=[
            pltpu.VMEM((n_chunks, ch), jnp.int32),
            pltpu.VMEM((ch, Dd), table.dtype),
            pltpu.SemaphoreType.DMA,
        ],
    )
    def k(table_hbm, idx_hbm, out_hbm, idx_v, rows_v, sem):
        wid = lax.axis_index("s") * NC + lax.axis_index("c")
        base = wid * b_per_w
        for c in range(n_chunks):
            pltpu.sync_copy(idx_hbm.at[pl.ds(base + c * ch, ch)], idx_v.at[c])
            pltpu.async_copy(table_hbm.at[idx_v.at[c]], rows_v, sem).wait()
            pltpu.sync_copy(rows_v, out_hbm.at[pl.ds(base + c * ch, ch)])

    return k(table, idx)


def _tc_gmm(xs, w1, w2, w3, be32):
    def body(
        be_r,
        x_ref, w1_hbm, w3_hbm, w2_hbm, o_ref,
        w1b, w3b, w2b, sems,
    ):
        b = pl.program_id(0)
        nchg = lax.fori_loop(
            0,
            NB,
            lambda i, acc: acc
            + jnp.where((i >= 1) & (i <= b) & (be_r[i] != be_r[i - 1]), 1, 0),
            0,
        )
        s = nchg % 2
        chg = jnp.where(b == 0, 1, jnp.where(be_r[b] != be_r[jnp.maximum(b - 1, 0)], 1, 0))
        nxtidx = lax.fori_loop(
            0,
            NB,
            lambda i, acc: jnp.where(
                (i > b) & (be_r[i] != be_r[b]) & (acc >= NB), i, acc
            ),
            NB,
        )
        hn = jnp.where(nxtidx < NB, 1, 0)
        nxt_e = be_r[jnp.minimum(nxtidx, NB - 1)]

        def issue(e, sl):
            pltpu.make_async_copy(w1_hbm.at[e], w1b.at[sl], sems.at[sl]).start()
            pltpu.make_async_copy(w3_hbm.at[e], w3b.at[sl], sems.at[sl]).start()
            pltpu.make_async_copy(w2_hbm.at[e], w2b.at[sl], sems.at[sl]).start()

        def wait(sl):
            pltpu.make_async_copy(w1_hbm.at[0], w1b.at[sl], sems.at[sl]).wait()
            pltpu.make_async_copy(w3_hbm.at[0], w3b.at[sl], sems.at[sl]).wait()
            pltpu.make_async_copy(w2_hbm.at[0], w2b.at[sl], sems.at[sl]).wait()

        @pl.when(b == 0)
        def _():
            issue(be_r[0], s)

        @pl.when(chg == 1)
        def _():
            wait(s)

            @pl.when(hn == 1)
            def _():
                issue(nxt_e, 1 - s)

        x = x_ref[...].astype(jnp.bfloat16)
        h = jnp.dot(x, w1b[s].astype(jnp.bfloat16), preferred_element_type=jnp.float32)
        g = jnp.dot(x, w3b[s].astype(jnp.bfloat16), preferred_element_type=jnp.float32)
        a = (h * jax.nn.sigmoid(h) * g).astype(jnp.bfloat16)
        o_ref[...] = jnp.dot(a, w2b[s].astype(jnp.bfloat16), preferred_element_type=jnp.float32)

    grid_spec = pltpu.PrefetchScalarGridSpec(
        num_scalar_prefetch=1,
        grid=(NB,),
        in_specs=[
            pl.BlockSpec((T, D), lambda b, *_: (b, 0)),
            pl.BlockSpec(memory_space=pl.ANY),
            pl.BlockSpec(memory_space=pl.ANY),
            pl.BlockSpec(memory_space=pl.ANY),
        ],
        out_specs=pl.BlockSpec((T, D), lambda b, *_: (b, 0)),
        scratch_shapes=[
            pltpu.VMEM((2, D, F), jnp.float32),
            pltpu.VMEM((2, D, F), jnp.float32),
            pltpu.VMEM((2, F, D), jnp.float32),
            pltpu.SemaphoreType.DMA((2,)),
        ],
    )
    return pl.pallas_call(
        body,
        grid_spec=grid_spec,
        out_shape=jax.ShapeDtypeStruct((NP, D), jnp.float32),
    )(be32[:NB], xs, w1, w3, w2)


def _sc_route_dispatch(hidden_states, top_flat):
    MK = M * K
    jpw = MK // NW
    half = jpw // 2
    nvr = MK // 16
    vpw = jpw // 16
    mesh = plsc.VectorSubcoreMesh(
        core_axis_name="c", subcore_axis_name="s", num_cores=NC, num_subcores=NS
    )

    @functools.partial(
        pl.kernel,
        out_type=(
            jax.ShapeDtypeStruct((NP, D), jnp.float32),
            jax.ShapeDtypeStruct((MK,), jnp.int32),
            jax.ShapeDtypeStruct((32,), jnp.int32),
        ),
        mesh=mesh,
        scratch_types=[
            pltpu.VMEM((MK,), jnp.int32),
            pltpu.VMEM((jpw,), jnp.int32),
            pltpu.VMEM((2, half), jnp.int32),
            pltpu.VMEM((2, half), jnp.int32),
            pltpu.VMEM((half, D), jnp.float32),
            pltpu.SemaphoreType.DMA,
            pltpu.SemaphoreType.DMA,
        ],
    )
    def k(hs, tf, disp, pos, be, ids, stage, idxg, idxs, rows, sem1, sem2):
        wid = lax.axis_index("s") * NC + lax.axis_index("c")
        pltpu.sync_copy(tf, ids)
        iota = lax.iota(jnp.int32, 16)
        lane15 = jnp.full((16,), 15, jnp.int32)
        zero16 = jnp.zeros((16,), jnp.int32)

        gdn = lax.GatherDimensionNumbers(
            offset_dims=(), collapsed_slice_dims=(0,), start_index_map=(0,)
        )

        def splat(vec, idxv):
            return lax.gather(
                vec,
                idxv.reshape(16, 1),
                gdn,
                (1,),
                mode=lax.GatherScatterMode.PROMISE_IN_BOUNDS,
            )

        one16 = jnp.full((16,), 1, jnp.int32)

        def butterfly_sum(x):
            for s in (1, 2, 4, 8):
                x = x + splat(x, jnp.bitwise_xor(iota, s))
            return x

        def vcumsum(x):
            for s in (1, 2, 4, 8):
                shifted = splat(x, jnp.maximum(iota - s, 0))
                x = x + jnp.where(iota >= s, shifted, zero16)
            return x

        def chunk_hist(w0, acc):
            def pb(j, accs):
                a1, a2 = accs
                v = ids[pl.ds((w0 * vpw + j) * 16, 16)]
                sh = jnp.left_shift(one16, (v & 3) * 8)
                a1 = a1 + jnp.where(v < 4, sh, zero16)
                a2 = a2 + jnp.where(v >= 4, sh, zero16)
                return a1, a2

            a1, a2 = lax.fori_loop(0, vpw, pb, (zero16, zero16))
            t1 = butterfly_sum(a1)
            t2 = butterfly_sum(a2)
            for e in range(4):
                c1 = jnp.bitwise_and(jnp.right_shift(t1, e * 8), 255)
                c2 = jnp.bitwise_and(jnp.right_shift(t2, e * 8), 255)
                acc = (
                    acc
                    + jnp.where(iota == e, c1, zero16)
                    + jnp.where(iota == e + 4, c2, zero16)
                )
            return acc

        tot = lax.fori_loop(0, NW, chunk_hist, zero16)
        prior = lax.fori_loop(0, wid, chunk_hist, zero16)

        padded = jnp.bitwise_and(tot + (T - 1), -T)
        offs_p = vcumsum(padded) - padded
        base_vec = offs_p + prior

        @pl.when(wid == 0)
        def _():
            for r in range(2):
                bstart = (iota + 16 * r) * T
                accb = jnp.full((16,), -1, jnp.int32)
                for e in range(E):
                    offe = splat(offs_p, jnp.full((16,), e, jnp.int32))
                    accb = accb + jnp.where(bstart >= offe, 1, 0)
                stage[pl.ds(16 * r, 16)] = accb
            pltpu.sync_copy(stage.at[pl.ds(0, 32)], be)

        for i in range(vpw):
            v = ids[pl.ds(wid * jpw + 16 * i, 16)]
            dest = jnp.zeros((16,), jnp.int32)
            for e in range(E):
                m = v == e
                cs = vcumsum(jnp.where(m, 1, 0))
                bse = splat(base_vec, jnp.full((16,), e, jnp.int32))
                dest = jnp.where(m, bse + cs - 1, dest)
                tot_splat = splat(cs, lane15)
                base_vec = base_vec + jnp.where(iota == e, tot_splat, zero16)
            stage[pl.ds(16 * i, 16)] = dest
            idxs[i // (vpw // 2), pl.ds((i % (vpw // 2)) * 16, 16)] = dest
            idxg[i // (vpw // 2), pl.ds((i % (vpw // 2)) * 16, 16)] = jnp.right_shift(
                wid * jpw + 16 * i + iota, 1
            )
        pltpu.sync_copy(stage, pos.at[pl.ds(wid * jpw, jpw)])

        for c2 in range(2):
            pltpu.async_copy(hs.at[idxg.at[c2]], rows, sem1).wait()
            pltpu.async_copy(rows, disp.at[idxs.at[c2]], sem2).wait()

    return k(hidden_states, top_flat)


def kernel(hidden_states, top_ks, w1, w2, w3):
    top_flat = top_ks.reshape(-1).astype(jnp.int32)
    disp, pos, be = _sc_route_dispatch(hidden_states, top_flat)
    ys = _tc_gmm(disp, w1, w2, w3, be)
    out = _sc_gather(ys, pos, n_chunks=2)
    return out.reshape(M, K, D)


# speedup vs baseline: 1.8732x; 1.0124x over previous
import functools

import jax
import jax.numpy as jnp
from jax import lax
from jax.experimental import pallas as pl
from jax.experimental.pallas import tpu as pltpu
from jax.experimental.pallas import tpu_sc as plsc

E = 8
K = 2
D = 1024
F = 2048
M = 2048

T = 256
NP = 6144
NB = NP // T

NC = 2
NS = 16
NW = NC * NS


def _sc_gather(table, idx, n_chunks):
    R, Dd = table.shape
    B = idx.shape[0]
    b_per_w = B // NW
    ch = b_per_w // n_chunks
    mesh = plsc.VectorSubcoreMesh(
        core_axis_name="c", subcore_axis_name="s", num_cores=NC, num_subcores=NS
    )

    @functools.partial(
        pl.kernel,
        out_type=jax.ShapeDtypeStruct((B, Dd), table.dtype),
        mesh=mesh,
        compiler_params=pltpu.CompilerParams(use_tc_tiling_on_sc=True),
        scratch_types=[
            pltpu.VMEM((n_chunks, ch), jnp.int32),
            pltpu.VMEM((ch, Dd), table.dtype),
            pltpu.SemaphoreType.DMA,
        ],
    )
    def k(table_hbm, idx_hbm, out_hbm, idx_v, rows_v, sem):
        wid = lax.axis_index("s") * NC + lax.axis_index("c")
        base = wid * b_per_w
        for c in range(n_chunks):
            pltpu.sync_copy(idx_hbm.at[pl.ds(base + c * ch, ch)], idx_v.at[c])
            pltpu.async_copy(table_hbm.at[idx_v.at[c]], rows_v, sem).wait()
            pltpu.sync_copy(rows_v, out_hbm.at[pl.ds(base + c * ch, ch)])

    return k(table, idx)


def _tc_gmm(xs, w1, w2, w3, be32):
    def body(
        be_r,
        x_ref, w1_hbm, w3_hbm, w2_hbm, o_ref,
        w1b, w3b, w2b, sems,
    ):
        b = pl.program_id(0)
        nchg = lax.fori_loop(
            0,
            NB,
            lambda i, acc: acc
            + jnp.where((i >= 1) & (i <= b) & (be_r[i] != be_r[i - 1]), 1, 0),
            0,
        )
        s = nchg % 2
        chg = jnp.where(b == 0, 1, jnp.where(be_r[b] != be_r[jnp.maximum(b - 1, 0)], 1, 0))
        nxtidx = lax.fori_loop(
            0,
            NB,
            lambda i, acc: jnp.where(
                (i > b) & (be_r[i] != be_r[b]) & (acc >= NB), i, acc
            ),
            NB,
        )
        hn = jnp.where(nxtidx < NB, 1, 0)
        nxt_e = be_r[jnp.minimum(nxtidx, NB - 1)]

        def issue(e, sl):
            pltpu.make_async_copy(w1_hbm.at[e], w1b.at[sl], sems.at[sl]).start()
            pltpu.make_async_copy(w3_hbm.at[e], w3b.at[sl], sems.at[sl]).start()
            pltpu.make_async_copy(w2_hbm.at[e], w2b.at[sl], sems.at[sl]).start()

        def wait(sl):
            pltpu.make_async_copy(w1_hbm.at[0], w1b.at[sl], sems.at[sl]).wait()
            pltpu.make_async_copy(w3_hbm.at[0], w3b.at[sl], sems.at[sl]).wait()
            pltpu.make_async_copy(w2_hbm.at[0], w2b.at[sl], sems.at[sl]).wait()

        @pl.when(b == 0)
        def _():
            issue(be_r[0], s)

        @pl.when(chg == 1)
        def _():
            wait(s)

            @pl.when(hn == 1)
            def _():
                issue(nxt_e, 1 - s)

        x = x_ref[...].astype(jnp.bfloat16)
        h = jnp.dot(x, w1b[s].astype(jnp.bfloat16), preferred_element_type=jnp.float32)
        g = jnp.dot(x, w3b[s].astype(jnp.bfloat16), preferred_element_type=jnp.float32)
        a = (h * jax.nn.sigmoid(h) * g).astype(jnp.bfloat16)
        o_ref[...] = jnp.dot(a, w2b[s].astype(jnp.bfloat16), preferred_element_type=jnp.float32)

    grid_spec = pltpu.PrefetchScalarGridSpec(
        num_scalar_prefetch=1,
        grid=(NB,),
        in_specs=[
            pl.BlockSpec((T, D), lambda b, *_: (b, 0)),
            pl.BlockSpec(memory_space=pl.ANY),
            pl.BlockSpec(memory_space=pl.ANY),
            pl.BlockSpec(memory_space=pl.ANY),
        ],
        out_specs=pl.BlockSpec((T, D), lambda b, *_: (b, 0)),
        scratch_shapes=[
            pltpu.VMEM((2, D, F), jnp.float32),
            pltpu.VMEM((2, D, F), jnp.float32),
            pltpu.VMEM((2, F, D), jnp.float32),
            pltpu.SemaphoreType.DMA((2,)),
        ],
    )
    return pl.pallas_call(
        body,
        grid_spec=grid_spec,
        out_shape=jax.ShapeDtypeStruct((NP, D), jnp.float32),
    )(be32[:NB], xs, w1, w3, w2)


def _sc_route_dispatch(hidden_states, top_flat):
    MK = M * K
    jpw = MK // NW
    half = jpw // 2
    nvr = MK // 16
    vpw = jpw // 16
    mesh = plsc.VectorSubcoreMesh(
        core_axis_name="c", subcore_axis_name="s", num_cores=NC, num_subcores=NS
    )

    @functools.partial(
        pl.kernel,
        out_type=(
            jax.ShapeDtypeStruct((NP, D), jnp.float32),
            jax.ShapeDtypeStruct((MK,), jnp.int32),
            jax.ShapeDtypeStruct((32,), jnp.int32),
        ),
        mesh=mesh,
        compiler_params=pltpu.CompilerParams(use_tc_tiling_on_sc=True),
        scratch_types=[
            pltpu.VMEM((MK,), jnp.int32),
            pltpu.VMEM((jpw,), jnp.int32),
            pltpu.VMEM((2, half), jnp.int32),
            pltpu.VMEM((2, half), jnp.int32),
            pltpu.VMEM((half, D), jnp.float32),
            pltpu.SemaphoreType.DMA,
            pltpu.SemaphoreType.DMA,
        ],
    )
    def k(hs, tf, disp, pos, be, ids, stage, idxg, idxs, rows, sem1, sem2):
        wid = lax.axis_index("s") * NC + lax.axis_index("c")
        pltpu.sync_copy(tf, ids)
        iota = lax.iota(jnp.int32, 16)
        lane15 = jnp.full((16,), 15, jnp.int32)
        zero16 = jnp.zeros((16,), jnp.int32)

        gdn = lax.GatherDimensionNumbers(
            offset_dims=(), collapsed_slice_dims=(0,), start_index_map=(0,)
        )

        def splat(vec, idxv):
            return lax.gather(
                vec,
                idxv.reshape(16, 1),
                gdn,
                (1,),
                mode=lax.GatherScatterMode.PROMISE_IN_BOUNDS,
            )

        one16 = jnp.full((16,), 1, jnp.int32)

        def butterfly_sum(x):
            for s in (1, 2, 4, 8):
                x = x + splat(x, jnp.bitwise_xor(iota, s))
            return x

        def vcumsum(x):
            for s in (1, 2, 4, 8):
                shifted = splat(x, jnp.maximum(iota - s, 0))
                x = x + jnp.where(iota >= s, shifted, zero16)
            return x

        def chunk_hist(w0, acc):
            def pb(j, accs):
                a1, a2 = accs
                v = ids[pl.ds((w0 * vpw + j) * 16, 16)]
                sh = jnp.left_shift(one16, (v & 3) * 8)
                a1 = a1 + jnp.where(v < 4, sh, zero16)
                a2 = a2 + jnp.where(v >= 4, sh, zero16)
                return a1, a2

            a1, a2 = lax.fori_loop(0, vpw, pb, (zero16, zero16))
            t1 = butterfly_sum(a1)
            t2 = butterfly_sum(a2)
            for e in range(4):
                c1 = jnp.bitwise_and(jnp.right_shift(t1, e * 8), 255)
                c2 = jnp.bitwise_and(jnp.right_shift(t2, e * 8), 255)
                acc = (
                    acc
                    + jnp.where(iota == e, c1, zero16)
                    + jnp.where(iota == e + 4, c2, zero16)
                )
            return acc

        tot = lax.fori_loop(0, NW, chunk_hist, zero16)
        prior = lax.fori_loop(0, wid, chunk_hist, zero16)

        padded = jnp.bitwise_and(tot + (T - 1), -T)
        offs_p = vcumsum(padded) - padded
        base_vec = offs_p + prior

        @pl.when(wid == 0)
        def _():
            for r in range(2):
                bstart = (iota + 16 * r) * T
                accb = jnp.full((16,), -1, jnp.int32)
                for e in range(E):
                    offe = splat(offs_p, jnp.full((16,), e, jnp.int32))
                    accb = accb + jnp.where(bstart >= offe, 1, 0)
                stage[pl.ds(16 * r, 16)] = accb
            pltpu.sync_copy(stage.at[pl.ds(0, 32)], be)

        for i in range(vpw):
            v = ids[pl.ds(wid * jpw + 16 * i, 16)]
            dest = jnp.zeros((16,), jnp.int32)
            for e in range(E):
                m = v == e
                cs = vcumsum(jnp.where(m, 1, 0))
                bse = splat(base_vec, jnp.full((16,), e, jnp.int32))
                dest = jnp.where(m, bse + cs - 1, dest)
                tot_splat = splat(cs, lane15)
                base_vec = base_vec + jnp.where(iota == e, tot_splat, zero16)
            stage[pl.ds(16 * i, 16)] = dest
            idxs[i // (vpw // 2), pl.ds((i % (vpw // 2)) * 16, 16)] = dest
            idxg[i // (vpw // 2), pl.ds((i % (vpw // 2)) * 16, 16)] = jnp.right_shift(
                wid * jpw + 16 * i + iota, 1
            )
        pltpu.sync_copy(stage, pos.at[pl.ds(wid * jpw, jpw)])

        for c2 in range(2):
            pltpu.async_copy(hs.at[idxg.at[c2]], rows, sem1).wait()
            pltpu.async_copy(rows, disp.at[idxs.at[c2]], sem2).wait()

    return k(hidden_states, top_flat)


def kernel(hidden_states, top_ks, w1, w2, w3):
    top_flat = top_ks.reshape(-1).astype(jnp.int32)
    disp, pos, be = _sc_route_dispatch(hidden_states, top_flat)
    ys = _tc_gmm(disp, w1, w2, w3, be)
    out = _sc_gather(ys, pos, n_chunks=2)
    return out.reshape(M, K, D)
